# Initial kernel scaffold; baseline (speedup 1.0000x reference)
import jax
import jax.numpy as jnp
from jax.experimental import pallas as pl


def kernel(x, gate_w, classifier_w):
    raise NotImplementedError("write your pallas kernel here")


# speedup vs baseline: 3.0969x; 3.0969x over previous
import jax
import jax.numpy as jnp
from jax import lax
from jax.experimental import pallas as pl

_HID = 4096
_TOPK = 8


def _rne_bf16_f32(v):
    u = lax.bitcast_convert_type(v, jnp.uint32)
    r = (u + jnp.uint32(0x7FFF) + ((u >> 16) & jnp.uint32(1))) & jnp.uint32(0xFFFF0000)
    return lax.bitcast_convert_type(r, jnp.float32)


def _router_kernel(x_ref, gw_ref, cw_ref, w_ref, i_ref):
    x = x_ref[...]
    g32 = jnp.dot(x, gw_ref[...], preferred_element_type=jnp.float32)
    c32 = jnp.dot(x, cw_ref[...], preferred_element_type=jnp.float32)
    g = _rne_bf16_f32(g32)
    one = jnp.float32(1)
    sig = one / (one + jnp.exp(-g))
    s = _rne_bf16_f32(jnp.abs(c32 * (g * sig)))

    col = lax.broadcasted_iota(jnp.int32, s.shape, 1)
    keys = (lax.bitcast_convert_type(s, jnp.int32) >> 4) | (_HID - 1 - col)

    mks = []
    for _ in range(_TOPK):
        mk = jnp.max(keys, axis=1, keepdims=True)
        mks.append(mk)
        keys = jnp.where(keys == mk, -1, keys)
    topk = jnp.concatenate(mks, axis=1)

    vals = lax.bitcast_convert_type((topk >> 12) << 16, jnp.float32)
    idxs = (_HID - 1) - (topk & (_HID - 1))

    m = vals[:, 0:1]
    denom = jnp.sum(jnp.exp(s - m), axis=1, keepdims=True)
    w_ref[...] = (jnp.exp(vals - m) / denom).astype(jnp.bfloat16)
    i_ref[...] = idxs


def kernel(x, gate_w, classifier_w):
    tokens = x.shape[0]
    t = 128
    weights, indices = pl.pallas_call(
        _router_kernel,
        grid=(tokens // t,),
        in_specs=[
            pl.BlockSpec((t, 64), lambda i: (i, 0)),
            pl.BlockSpec((64, _HID), lambda i: (0, 0)),
            pl.BlockSpec((64, _HID), lambda i: (0, 0)),
        ],
        out_specs=[
            pl.BlockSpec((t, _TOPK), lambda i: (i, 0)),
            pl.BlockSpec((t, _TOPK), lambda i: (i, 0)),
        ],
        out_shape=[
            jax.ShapeDtypeStruct((tokens, _TOPK), jnp.bfloat16),
            jax.ShapeDtypeStruct((tokens, _TOPK), jnp.int32),
        ],
    )(x, gate_w.T, classifier_w.T)
    return weights, indices


# speedup vs baseline: 4.5915x; 1.4826x over previous
import jax
import jax.numpy as jnp
from jax import lax
from jax.experimental import pallas as pl

_HID = 4096
_TOPK = 8


def _router_kernel(x_ref, gw_ref, cw_ref, w_ref, i_ref):
    x = x_ref[...]
    g32 = jnp.dot(x, gw_ref[...], preferred_element_type=jnp.float32)
    c32 = jnp.dot(x, cw_ref[...], preferred_element_type=jnp.float32)
    g = g32.astype(jnp.bfloat16).astype(jnp.float32)
    one = jnp.float32(1)
    sig = one / (one + jnp.exp(-g))
    s16 = jnp.abs(c32 * (g * sig)).astype(jnp.bfloat16)
    s = s16.astype(jnp.float32)

    col = lax.broadcasted_iota(jnp.int32, s.shape, 1)
    kb = (lax.bitcast_convert_type(s, jnp.int32) >> 4) | (_HID - 1 - col)
    keys = lax.bitcast_convert_type(kb | jnp.int32(0x40000000), jnp.float32)

    mks = []
    for _ in range(_TOPK):
        mk = jnp.max(keys, axis=1, keepdims=True)
        mks.append(mk)
        keys = jnp.where(keys == mk, jnp.float32(0), keys)
    topk = lax.bitcast_convert_type(jnp.concatenate(mks, axis=1), jnp.int32)

    vals = lax.bitcast_convert_type(
        (topk << 4) & jnp.int32(0xFFFF0000 - 0x100000000), jnp.float32)
    idxs = (_HID - 1) - (topk & (_HID - 1))

    m = vals[:, 0:1]
    denom = jnp.sum(jnp.exp(s - m), axis=1, keepdims=True)
    w_ref[...] = (jnp.exp(vals - m) / denom).astype(jnp.bfloat16)
    i_ref[...] = idxs


def kernel(x, gate_w, classifier_w):
    tokens = x.shape[0]
    t = 128
    weights, indices = pl.pallas_call(
        _router_kernel,
        grid=(tokens // t,),
        in_specs=[
            pl.BlockSpec((t, 64), lambda i: (i, 0)),
            pl.BlockSpec((64, _HID), lambda i: (0, 0)),
            pl.BlockSpec((64, _HID), lambda i: (0, 0)),
        ],
        out_specs=[
            pl.BlockSpec((t, _TOPK), lambda i: (i, 0)),
            pl.BlockSpec((t, _TOPK), lambda i: (i, 0)),
        ],
        out_shape=[
            jax.ShapeDtypeStruct((tokens, _TOPK), jnp.bfloat16),
            jax.ShapeDtypeStruct((tokens, _TOPK), jnp.int32),
        ],
    )(x, gate_w.T, classifier_w.T)
    return weights, indices


# speedup vs baseline: 5.1250x; 1.1162x over previous
import jax
import jax.numpy as jnp
from jax import lax
from jax.experimental import pallas as pl

_HID = 4096
_TOPK = 8


def _router_kernel(x_ref, gw_ref, cw_ref, w_ref, i_ref):
    x = x_ref[...]
    g32 = jnp.dot(x, gw_ref[...], preferred_element_type=jnp.float32)
    c32 = jnp.dot(x, cw_ref[...], preferred_element_type=jnp.float32)
    g = g32.astype(jnp.bfloat16).astype(jnp.float32)
    one = jnp.float32(1)
    sig = one / (one + jnp.exp(-g))
    s16 = jnp.abs(c32 * (g * sig)).astype(jnp.bfloat16)
    s = s16.astype(jnp.float32)

    col = lax.broadcasted_iota(jnp.int32, s.shape, 1)
    kb = (lax.bitcast_convert_type(s, jnp.int32) >> 4) | (_HID - 1 - col)
    keys = lax.bitcast_convert_type(kb | jnp.int32(0x40000000), jnp.float32)

    w8 = _HID // _TOPK
    r = [keys[:, i * w8:(i + 1) * w8] for i in range(_TOPK)]
    net = [(0, 1), (2, 3), (4, 5), (6, 7), (0, 2), (1, 3), (4, 6), (5, 7),
           (1, 2), (5, 6), (0, 4), (3, 7), (1, 5), (2, 6), (1, 4), (3, 6),
           (2, 4), (3, 5), (3, 4)]
    for a, b in net:
        hi = jnp.maximum(r[a], r[b])
        lo = jnp.minimum(r[a], r[b])
        r[a], r[b] = hi, lo

    mks = []
    for _ in range(_TOPK):
        mk = jnp.max(r[0], axis=1, keepdims=True)
        mks.append(mk)
        cond = r[0] == mk
        for i in range(_TOPK - 1):
            r[i] = jnp.where(cond, r[i + 1], r[i])
        r[_TOPK - 1] = jnp.where(cond, jnp.float32(0), r[_TOPK - 1])
    topk = lax.bitcast_convert_type(jnp.concatenate(mks, axis=1), jnp.int32)

    vals = lax.bitcast_convert_type(
        (topk << 4) & jnp.int32(0xFFFF0000 - 0x100000000), jnp.float32)
    idxs = (_HID - 1) - (topk & (_HID - 1))

    m = vals[:, 0:1]
    denom = jnp.sum(jnp.exp(s - m), axis=1, keepdims=True)
    w_ref[...] = (jnp.exp(vals - m) / denom).astype(jnp.bfloat16)
    i_ref[...] = idxs


def kernel(x, gate_w, classifier_w):
    tokens = x.shape[0]
    t = 128
    weights, indices = pl.pallas_call(
        _router_kernel,
        grid=(tokens // t,),
        in_specs=[
            pl.BlockSpec((t, 64), lambda i: (i, 0)),
            pl.BlockSpec((64, _HID), lambda i: (0, 0)),
            pl.BlockSpec((64, _HID), lambda i: (0, 0)),
        ],
        out_specs=[
            pl.BlockSpec((t, _TOPK), lambda i: (i, 0)),
            pl.BlockSpec((t, _TOPK), lambda i: (i, 0)),
        ],
        out_shape=[
            jax.ShapeDtypeStruct((tokens, _TOPK), jnp.bfloat16),
            jax.ShapeDtypeStruct((tokens, _TOPK), jnp.int32),
        ],
    )(x, gate_w.T, classifier_w.T)
    return weights, indices
